# baseline (speedup 1.0000x reference)
import functools

import jax
import jax.numpy as jnp
from jax import lax
from jax.experimental import pallas as pl
from jax.experimental.pallas import tpu as pltpu
from jax.experimental.pallas import tpu_sc as plsc

N = 10000
E = 320000
G = 64
H = 128

NC = 2
NS = 16
NW = NC * NS
EPW = E // NW
CHUNK = 100
NCHUNK = EPW // CHUNK
RPT = 640
RPT_LAST = N - (NS - 1) * RPT
STRIP = 40
CW = 16
NBUF = 2

_MESH = plsc.VectorSubcoreMesh(
    core_axis_name="c", subcore_axis_name="s", num_cores=NC,
    num_subcores=NS)
_SC_PARAMS = pltpu.CompilerParams(use_tc_tiling_on_sc=False)


def _strip_loop(nrow, fn):
  for t in range(nrow // STRIP):
    fn(t * STRIP)


def _per_tile(sid, fn):
  @pl.when(sid != NS - 1)
  def _():
    fn(RPT)

  @pl.when(sid == NS - 1)
  def _():
    fn(RPT_LAST)


def _make_edge_agg(with_counts):
  scratch = [
      pltpu.VMEM((STRIP, H), jnp.bfloat16),
      pltpu.VMEM((NCHUNK, CHUNK), jnp.int32),
      pltpu.VMEM((NCHUNK, CHUNK), jnp.int32),
      pltpu.VMEM_SHARED((N, H), jnp.bfloat16),
  ] + [pltpu.VMEM((CHUNK, H), jnp.bfloat16) for _ in range(NBUF)] \
    + [pltpu.SemaphoreType.DMA for _ in range(NBUF)]
  out_type = [jax.ShapeDtypeStruct((NC * N, H), jnp.bfloat16)]
  if with_counts:
    out_type.append(jax.ShapeDtypeStruct((NC * N, CW), jnp.bfloat16))
    scratch += [
        pltpu.VMEM((STRIP, CW), jnp.bfloat16),
        pltpu.VMEM((CHUNK, CW), jnp.bfloat16),
        pltpu.VMEM_SHARED((N, CW), jnp.bfloat16),
    ]

  def body(h_hbm, src_hbm, dst_hbm, z_rows, z_cnt, ones_hbm, *rest):
    if with_counts:
      acc_out, cnt_out, zbuf, sbuf, dbuf, acc_sh, *rest2 = rest
      rows = rest2[:NBUF]
      sems = rest2[NBUF:2 * NBUF]
      cbuf, ones_v, cnt_sh = rest2[2 * NBUF:]
    else:
      acc_out, zbuf, sbuf, dbuf, acc_sh, *rest2 = rest
      rows = rest2[:NBUF]
      sems = rest2[NBUF:2 * NBUF]
    cid = lax.axis_index("c")
    sid = lax.axis_index("s")
    wid = cid * NS + sid

    def zero_part(nrow):
      pltpu.sync_copy(z_rows, zbuf)
      if with_counts:
        pltpu.sync_copy(z_cnt, cbuf)
      for t in range(nrow // STRIP):
        pltpu.sync_copy(zbuf, acc_sh.at[pl.ds(sid * RPT + t * STRIP,
                                              STRIP)])
        if with_counts:
          pltpu.sync_copy(cbuf, cnt_sh.at[pl.ds(sid * RPT + t * STRIP,
                                                STRIP)])

    _per_tile(sid, zero_part)

    pltpu.sync_copy(src_hbm.at[wid], sbuf)
    pltpu.sync_copy(dst_hbm.at[wid], dbuf)
    if with_counts:
      pltpu.sync_copy(ones_hbm, ones_v)
    plsc.subcore_barrier()

    for b in range(NBUF):
      pltpu.async_copy(h_hbm.at[sbuf.at[b]], rows[b], sems[b])

    def outer(it, carry):
      j0 = it * NBUF
      for b in range(NBUF):
        j = j0 + b
        pltpu.make_async_copy(h_hbm.at[sbuf.at[0]], rows[b],
                              sems[b]).wait()
        pltpu.sync_copy(rows[b], acc_sh.at[dbuf.at[j]], add=True)
        if with_counts:
          pltpu.sync_copy(ones_v, cnt_sh.at[dbuf.at[j]], add=True)

        @pl.when(j < NCHUNK - NBUF)
        def _():
          pltpu.async_copy(h_hbm.at[sbuf.at[j + NBUF]], rows[b], sems[b])
      return carry

    lax.fori_loop(0, NCHUNK // NBUF, outer, 0)
    plsc.subcore_barrier()

    def write_part(nrow):
      for t in range(nrow // STRIP):
        r = t * STRIP
        pltpu.sync_copy(acc_sh.at[pl.ds(sid * RPT + r, STRIP)], zbuf)
        pltpu.sync_copy(zbuf, acc_out.at[pl.ds(cid * N + sid * RPT + r,
                                               STRIP)])
        if with_counts:
          pltpu.sync_copy(cnt_sh.at[pl.ds(sid * RPT + r, STRIP)], cbuf)
          pltpu.sync_copy(cbuf, cnt_out.at[pl.ds(cid * N + sid * RPT + r,
                                                 STRIP)])

    _per_tile(sid, write_part)

  return pl.kernel(body, out_type=out_type, mesh=_MESH,
                   scratch_types=scratch, compiler_params=_SC_PARAMS)


_edge_agg_cnt = _make_edge_agg(True)
_edge_agg = _make_edge_agg(False)


def _lin1_body(x_ref, w_ref, b_ref, o_ref):
  out = lax.dot_general(
      x_ref[...], w_ref[...], (((1,), (1,)), ((), ())),
      preferred_element_type=jnp.float32) + b_ref[...]
  o_ref[...] = out.astype(jnp.bfloat16)


def _sage_combine(pal_ref, par_ref, pcl_ref, pcr_ref, h_ref, wl_ref,
                  bl_ref, wr_ref):
  agg = pal_ref[...].astype(jnp.float32) + par_ref[...].astype(jnp.float32)
  cnt = (pcl_ref[:, 0:1].astype(jnp.float32)
         + pcr_ref[:, 0:1].astype(jnp.float32))
  mean = agg / jnp.maximum(cnt, 1.0)
  h = h_ref[...].astype(jnp.float32)
  out = lax.dot_general(mean, wl_ref[...], (((1,), (1,)), ((), ())),
                        preferred_element_type=jnp.float32)
  out = out + bl_ref[...]
  out = out + lax.dot_general(h, wr_ref[...], (((1,), (1,)), ((), ())),
                              preferred_element_type=jnp.float32)
  norm = jnp.sqrt(jnp.sum(out * out, axis=1, keepdims=True))
  return out / jnp.maximum(norm, 1e-12)


def _combine_body(pal_ref, par_ref, pcl_ref, pcr_ref, h_ref, wl_ref,
                  bl_ref, wr_ref, o_ref):
  out = _sage_combine(pal_ref, par_ref, pcl_ref, pcr_ref, h_ref, wl_ref,
                      bl_ref, wr_ref)
  o_ref[...] = jnp.maximum(out, 0.0).astype(jnp.bfloat16)


def _final_body(nsteps, pal_ref, par_ref, pcl_ref, pcr_ref, h_ref, wl_ref,
                bl_ref, wr_ref, batch_ref, wout_ref, bout_ref, o_ref,
                gsum, gcnt):
  i = pl.program_id(0)
  h3 = _sage_combine(pal_ref, par_ref, pcl_ref, pcr_ref, h_ref, wl_ref,
                     bl_ref, wr_ref)

  bn = h3.shape[0]
  gid = lax.broadcasted_iota(jnp.int32, (G, bn), 0)
  onehot = jnp.where(gid == batch_ref[0], 1.0, 0.0)

  @pl.when(i == 0)
  def _():
    gsum[...] = jnp.zeros_like(gsum)
    gcnt[...] = jnp.zeros_like(gcnt)

  gsum[...] += lax.dot_general(onehot, h3, (((1,), (0,)), ((), ())),
                               preferred_element_type=jnp.float32)
  gcnt[...] += jnp.sum(onehot, axis=1, keepdims=True)

  @pl.when(i == nsteps - 1)
  def _():
    pooled = gsum[...] / jnp.maximum(gcnt[...], 1.0)
    logits = jnp.sum(pooled * wout_ref[...], axis=1, keepdims=True)
    o_ref[...] = jax.nn.sigmoid(logits + bout_ref[0, 0])


def kernel(x, edge_index, batch, W1, b1, c1_Wl, c1_bl, c1_Wr, c2_Wl,
           c2_bl, c2_Wr, Wout, bout):
  D = x.shape[1]
  H2 = c2_Wl.shape[0]
  src = edge_index[0]
  dst = edge_index[1]

  BN = N
  nsteps = N // BN
  row_spec = pl.BlockSpec((BN, H), lambda i: (i, 0))
  left_spec = pl.BlockSpec((BN, H), lambda i: (i, 0))
  right_spec = pl.BlockSpec((BN, H), lambda i: (i + nsteps, 0))
  cntl_spec = pl.BlockSpec((BN, CW), lambda i: (i, 0))
  cntr_spec = pl.BlockSpec((BN, CW), lambda i: (i + nsteps, 0))
  w_spec = lambda r, c: pl.BlockSpec((r, c), lambda i: (0, 0))

  src3 = src.reshape(NW, NCHUNK, CHUNK)
  dst3 = dst.reshape(NW, NCHUNK, CHUNK)
  z_rows = jnp.zeros((STRIP, H), jnp.bfloat16)
  z_cnt = jnp.zeros((STRIP, CW), jnp.bfloat16)
  ones = jnp.ones((CHUNK, CW), jnp.bfloat16)

  (h,) = pl.pallas_call(
      _lin1_body,
      grid=(nsteps,),
      in_specs=[
          pl.BlockSpec((BN, D), lambda i: (i, 0)),
          pl.BlockSpec((H, D), lambda i: (0, 0)),
          pl.BlockSpec((1, H), lambda i: (0, 0)),
      ],
      out_specs=[row_spec],
      out_shape=[jax.ShapeDtypeStruct((N, H), jnp.bfloat16)],
  )(x, W1, b1.reshape(1, H))

  agg1, cnt = _edge_agg_cnt(h, src3, dst3, z_rows, z_cnt, ones)

  (h2,) = pl.pallas_call(
      _combine_body,
      grid=(nsteps,),
      in_specs=[
          left_spec, right_spec, cntl_spec, cntr_spec, row_spec,
          w_spec(H, H), w_spec(1, H), w_spec(H, H),
      ],
      out_specs=[row_spec],
      out_shape=[jax.ShapeDtypeStruct((N, H), jnp.bfloat16)],
  )(agg1, agg1, cnt, cnt, h, c1_Wl, c1_bl.reshape(1, H), c1_Wr)

  (agg2,) = _edge_agg(h2, src3, dst3, z_rows, z_cnt, ones)

  batchi = batch.reshape(nsteps, 1, BN)
  out = pl.pallas_call(
      functools.partial(_final_body, nsteps),
      grid=(nsteps,),
      in_specs=[
          left_spec, right_spec, cntl_spec, cntr_spec, row_spec,
          w_spec(H2, H), w_spec(1, H2), w_spec(H2, H),
          pl.BlockSpec((1, 1, BN), lambda i: (i, 0, 0)),
          w_spec(1, H2), w_spec(1, 1),
      ],
      out_specs=pl.BlockSpec((G, 1), lambda i: (0, 0)),
      out_shape=jax.ShapeDtypeStruct((G, 1), jnp.float32),
      scratch_shapes=[
          pltpu.VMEM((G, H2), jnp.float32),
          pltpu.VMEM((G, 1), jnp.float32),
      ],
  )(agg2, agg2, cnt, cnt, h2, c2_Wl, c2_bl.reshape(1, H2), c2_Wr,
    batchi, Wout, bout.reshape(1, 1))
  return out


# speedup vs baseline: 1.2481x; 1.2481x over previous
import functools

import jax
import jax.numpy as jnp
from jax import lax
from jax.experimental import pallas as pl
from jax.experimental.pallas import tpu as pltpu
from jax.experimental.pallas import tpu_sc as plsc

N = 10000
E = 320000
G = 64
H = 128

NC = 2
NS = 16
NW = NC * NS
EPW = E // NW
CHUNK = 80
NCHUNK = EPW // CHUNK
CPT = E // NS // CHUNK
CHALF = CPT // 2
RPT = 640
RPT_LAST = N - (NS - 1) * RPT
STRIP = 80
CW = 16
NBUF = 5

_MESH = plsc.VectorSubcoreMesh(
    core_axis_name="c", subcore_axis_name="s", num_cores=NC,
    num_subcores=NS)
_SC_PARAMS = pltpu.CompilerParams(use_tc_tiling_on_sc=False)


def _strip_loop(nrow, fn):
  for t in range(nrow // STRIP):
    fn(t * STRIP)


def _per_tile(sid, fn):
  @pl.when(sid != NS - 1)
  def _():
    fn(RPT)

  @pl.when(sid == NS - 1)
  def _():
    fn(RPT_LAST)


def _agg_body(h_hbm, src_hbm, dst_hbm, z_rows, acc_out, zbuf, sbuf, dbuf,
              acc_sh, *rest):
  rows = rest[:NBUF]
  sems = rest[NBUF:]
  cid = lax.axis_index("c")
  sid = lax.axis_index("s")
  wid = cid * NS + sid

  def zero_part(nrow):
    pltpu.sync_copy(z_rows, zbuf)
    _strip_loop(nrow, lambda r: pltpu.sync_copy(
        zbuf, acc_sh.at[pl.ds(sid * RPT + r, STRIP)]))

  _per_tile(sid, zero_part)

  pltpu.sync_copy(src_hbm.at[wid], sbuf)
  pltpu.sync_copy(dst_hbm.at[wid], dbuf)
  plsc.subcore_barrier()

  for b in range(NBUF):
    pltpu.async_copy(h_hbm.at[sbuf.at[b]], rows[b], sems[b])

  def outer(it, carry):
    j0 = it * NBUF
    for b in range(NBUF):
      j = j0 + b
      pltpu.make_async_copy(h_hbm.at[sbuf.at[0]], rows[b], sems[b]).wait()
      pltpu.sync_copy(rows[b], acc_sh.at[dbuf.at[j]], add=True)

      @pl.when(j < NCHUNK - NBUF)
      def _():
        pltpu.async_copy(h_hbm.at[sbuf.at[j + NBUF]], rows[b], sems[b])
    return carry

  lax.fori_loop(0, NCHUNK // NBUF, outer, 0)
  plsc.subcore_barrier()

  def write_part(nrow):
    def strip(r):
      pltpu.sync_copy(acc_sh.at[pl.ds(sid * RPT + r, STRIP)], zbuf)
      pltpu.sync_copy(zbuf, acc_out.at[pl.ds(cid * N + sid * RPT + r,
                                             STRIP)])
    _strip_loop(nrow, strip)

  _per_tile(sid, write_part)


_edge_agg = pl.kernel(
    _agg_body,
    out_type=[jax.ShapeDtypeStruct((NC * N, H), jnp.bfloat16)],
    mesh=_MESH,
    scratch_types=[
        pltpu.VMEM((STRIP, H), jnp.bfloat16),
        pltpu.VMEM((NCHUNK, CHUNK), jnp.int32),
        pltpu.VMEM((NCHUNK, CHUNK), jnp.int32),
        pltpu.VMEM_SHARED((N, H), jnp.bfloat16),
    ] + [pltpu.VMEM((CHUNK, H), jnp.bfloat16) for _ in range(NBUF)]
      + [pltpu.SemaphoreType.DMA for _ in range(NBUF)],
    compiler_params=_SC_PARAMS,
)


def _count_body(dst_hbm, z_cnt, ones_hbm, cnt_out, cbuf, dbuf, ones_v,
                cnt_sh, sem):
  cid = lax.axis_index("c")
  sid = lax.axis_index("s")

  def zero_part(nrow):
    pltpu.sync_copy(z_cnt, cbuf)
    _strip_loop(nrow, lambda r: pltpu.sync_copy(
        cbuf, cnt_sh.at[pl.ds(sid * RPT + r, STRIP)]))

  _per_tile(sid, zero_part)
  pltpu.sync_copy(dst_hbm.at[sid], dbuf)
  pltpu.sync_copy(ones_hbm, ones_v)
  plsc.subcore_barrier()

  j0 = cid * CHALF

  def fire(j, carry):
    pltpu.async_copy(ones_v, cnt_sh.at[dbuf.at[j0 + j]], sem, add=True)
    return carry

  lax.fori_loop(0, CHALF, fire, 0)

  def drain(j, carry):
    pltpu.make_async_copy(ones_v, cnt_sh.at[dbuf.at[j0]], sem).wait()
    return carry

  lax.fori_loop(0, CHALF, drain, 0)
  plsc.subcore_barrier()

  def write_part(nrow):
    def strip(r):
      pltpu.sync_copy(cnt_sh.at[pl.ds(sid * RPT + r, STRIP)], cbuf)
      pltpu.sync_copy(cbuf, cnt_out.at[pl.ds(cid * N + sid * RPT + r,
                                             STRIP)])
    _strip_loop(nrow, strip)

  _per_tile(sid, write_part)


_edge_count = pl.kernel(
    _count_body,
    out_type=[jax.ShapeDtypeStruct((NC * N, CW), jnp.float32)],
    mesh=_MESH,
    scratch_types=[
        pltpu.VMEM((STRIP, CW), jnp.float32),
        pltpu.VMEM((CPT, CHUNK), jnp.int32),
        pltpu.VMEM((CHUNK, CW), jnp.float32),
        pltpu.VMEM_SHARED((N, CW), jnp.float32),
        pltpu.SemaphoreType.DMA,
    ],
    compiler_params=_SC_PARAMS,
)


def _lin1_body(x_ref, w_ref, b_ref, o_ref):
  out = lax.dot_general(
      x_ref[...], w_ref[...], (((1,), (1,)), ((), ())),
      preferred_element_type=jnp.float32) + b_ref[...]
  o_ref[...] = out.astype(jnp.bfloat16)


def _sage_combine(pal_ref, par_ref, pcl_ref, pcr_ref, h_ref, wl_ref,
                  bl_ref, wr_ref):
  agg = pal_ref[...].astype(jnp.float32) + par_ref[...].astype(jnp.float32)
  cnt = pcl_ref[:, 0:1] + pcr_ref[:, 0:1]
  mean = agg / jnp.maximum(cnt, 1.0)
  h = h_ref[...].astype(jnp.float32)
  out = lax.dot_general(mean, wl_ref[...], (((1,), (1,)), ((), ())),
                        preferred_element_type=jnp.float32)
  out = out + bl_ref[...]
  out = out + lax.dot_general(h, wr_ref[...], (((1,), (1,)), ((), ())),
                              preferred_element_type=jnp.float32)
  norm = jnp.sqrt(jnp.sum(out * out, axis=1, keepdims=True))
  return out / jnp.maximum(norm, 1e-12)


def _combine_body(pal_ref, par_ref, pcl_ref, pcr_ref, h_ref, wl_ref,
                  bl_ref, wr_ref, o_ref):
  out = _sage_combine(pal_ref, par_ref, pcl_ref, pcr_ref, h_ref, wl_ref,
                      bl_ref, wr_ref)
  o_ref[...] = jnp.maximum(out, 0.0).astype(jnp.bfloat16)


def _final_body(nsteps, pal_ref, par_ref, pcl_ref, pcr_ref, h_ref, wl_ref,
                bl_ref, wr_ref, batch_ref, wout_ref, bout_ref, o_ref,
                gsum, gcnt):
  i = pl.program_id(0)
  h3 = _sage_combine(pal_ref, par_ref, pcl_ref, pcr_ref, h_ref, wl_ref,
                     bl_ref, wr_ref)

  bn = h3.shape[0]
  gid = lax.broadcasted_iota(jnp.int32, (G, bn), 0)
  onehot = jnp.where(gid == batch_ref[0], 1.0, 0.0)

  @pl.when(i == 0)
  def _():
    gsum[...] = jnp.zeros_like(gsum)
    gcnt[...] = jnp.zeros_like(gcnt)

  gsum[...] += lax.dot_general(onehot, h3, (((1,), (0,)), ((), ())),
                               preferred_element_type=jnp.float32)
  gcnt[...] += jnp.sum(onehot, axis=1, keepdims=True)

  @pl.when(i == nsteps - 1)
  def _():
    pooled = gsum[...] / jnp.maximum(gcnt[...], 1.0)
    logits = jnp.sum(pooled * wout_ref[...], axis=1, keepdims=True)
    o_ref[...] = jax.nn.sigmoid(logits + bout_ref[0, 0])


def kernel(x, edge_index, batch, W1, b1, c1_Wl, c1_bl, c1_Wr, c2_Wl,
           c2_bl, c2_Wr, Wout, bout):
  D = x.shape[1]
  H2 = c2_Wl.shape[0]
  src = edge_index[0]
  dst = edge_index[1]

  BN = N
  nsteps = N // BN
  row_spec = pl.BlockSpec((BN, H), lambda i: (i, 0))
  left_spec = pl.BlockSpec((BN, H), lambda i: (i, 0))
  right_spec = pl.BlockSpec((BN, H), lambda i: (i + nsteps, 0))
  cntl_spec = pl.BlockSpec((BN, CW), lambda i: (i, 0))
  cntr_spec = pl.BlockSpec((BN, CW), lambda i: (i + nsteps, 0))
  w_spec = lambda r, c: pl.BlockSpec((r, c), lambda i: (0, 0))

  src3 = src.reshape(NW, NCHUNK, CHUNK)
  dst3 = dst.reshape(NW, NCHUNK, CHUNK)
  dstc = dst.reshape(NS, CPT, CHUNK)
  z_rows = jnp.zeros((STRIP, H), jnp.bfloat16)
  z_cnt = jnp.zeros((STRIP, CW), jnp.float32)
  ones = jnp.ones((CHUNK, CW), jnp.float32)

  (cnt,) = _edge_count(dstc, z_cnt, ones)

  (h,) = pl.pallas_call(
      _lin1_body,
      grid=(nsteps,),
      in_specs=[
          pl.BlockSpec((BN, D), lambda i: (i, 0)),
          pl.BlockSpec((H, D), lambda i: (0, 0)),
          pl.BlockSpec((1, H), lambda i: (0, 0)),
      ],
      out_specs=[row_spec],
      out_shape=[jax.ShapeDtypeStruct((N, H), jnp.bfloat16)],
  )(x, W1, b1.reshape(1, H))

  (agg1,) = _edge_agg(h, src3, dst3, z_rows)

  (h2,) = pl.pallas_call(
      _combine_body,
      grid=(nsteps,),
      in_specs=[
          left_spec, right_spec, cntl_spec, cntr_spec, row_spec,
          w_spec(H, H), w_spec(1, H), w_spec(H, H),
      ],
      out_specs=[row_spec],
      out_shape=[jax.ShapeDtypeStruct((N, H), jnp.bfloat16)],
  )(agg1, agg1, cnt, cnt, h, c1_Wl, c1_bl.reshape(1, H), c1_Wr)

  (agg2,) = _edge_agg(h2, src3, dst3, z_rows)

  batchi = batch.reshape(nsteps, 1, BN)
  out = pl.pallas_call(
      functools.partial(_final_body, nsteps),
      grid=(nsteps,),
      in_specs=[
          left_spec, right_spec, cntl_spec, cntr_spec, row_spec,
          w_spec(H2, H), w_spec(1, H2), w_spec(H2, H),
          pl.BlockSpec((1, 1, BN), lambda i: (i, 0, 0)),
          w_spec(1, H2), w_spec(1, 1),
      ],
      out_specs=pl.BlockSpec((G, 1), lambda i: (0, 0)),
      out_shape=jax.ShapeDtypeStruct((G, 1), jnp.float32),
      scratch_shapes=[
          pltpu.VMEM((G, H2), jnp.float32),
          pltpu.VMEM((G, 1), jnp.float32),
      ],
  )(agg2, agg2, cnt, cnt, h2, c2_Wl, c2_bl.reshape(1, H2), c2_Wr,
    batchi, Wout, bout.reshape(1, 1))
  return out


# speedup vs baseline: 1.2864x; 1.0307x over previous
import functools

import jax
import jax.numpy as jnp
from jax import lax
from jax.experimental import pallas as pl
from jax.experimental.pallas import tpu as pltpu
from jax.experimental.pallas import tpu_sc as plsc

N = 10000
E = 320000
G = 64
H = 128

NC = 2
NS = 16
NW = NC * NS
EPW = E // NW
CHUNK = 80
NCHUNK = EPW // CHUNK
CPT = E // NS // CHUNK
CHALF = CPT // 2
RPT = 640
RPT_LAST = N - (NS - 1) * RPT
STRIP = 80
CW = 16
NBUF = 5

_MESH = plsc.VectorSubcoreMesh(
    core_axis_name="c", subcore_axis_name="s", num_cores=NC,
    num_subcores=NS)
_SC_PARAMS = pltpu.CompilerParams(use_tc_tiling_on_sc=False)


def _strip_loop(nrow, fn):
  for t in range(nrow // STRIP):
    fn(t * STRIP)


def _per_tile(sid, fn):
  @pl.when(sid != NS - 1)
  def _():
    fn(RPT)

  @pl.when(sid == NS - 1)
  def _():
    fn(RPT_LAST)


def _agg_body(h_hbm, src_hbm, dst_hbm, z_rows, acc_out, zbuf, sbuf, dbuf,
              acc_sh, *rest):
  rows = rest[:NBUF]
  sems = rest[NBUF:]
  cid = lax.axis_index("c")
  sid = lax.axis_index("s")
  wid = cid * NS + sid

  pltpu.sync_copy(z_rows, zbuf)
  pltpu.async_copy(src_hbm.at[wid], sbuf, sems[0])
  pltpu.async_copy(dst_hbm.at[wid], dbuf, sems[1])

  def zero_part(nrow):
    ns = nrow // STRIP
    for t in range(ns):
      pltpu.async_copy(zbuf, acc_sh.at[pl.ds(sid * RPT + t * STRIP,
                                             STRIP)], sems[2])
    for _ in range(ns):
      pltpu.make_async_copy(zbuf, acc_sh.at[pl.ds(sid * RPT, STRIP)],
                            sems[2]).wait()

  _per_tile(sid, zero_part)
  pltpu.make_async_copy(src_hbm.at[wid], sbuf, sems[0]).wait()
  pltpu.make_async_copy(dst_hbm.at[wid], dbuf, sems[1]).wait()
  plsc.subcore_barrier()

  for b in range(NBUF):
    pltpu.async_copy(h_hbm.at[sbuf.at[b]], rows[b], sems[b])

  def outer(it, carry):
    j0 = it * NBUF
    for b in range(NBUF):
      j = j0 + b
      pltpu.make_async_copy(h_hbm.at[sbuf.at[0]], rows[b], sems[b]).wait()
      pltpu.sync_copy(rows[b], acc_sh.at[dbuf.at[j]], add=True)

      @pl.when(j < NCHUNK - NBUF)
      def _():
        pltpu.async_copy(h_hbm.at[sbuf.at[j + NBUF]], rows[b], sems[b])
    return carry

  lax.fori_loop(0, NCHUNK // NBUF, outer, 0)
  plsc.subcore_barrier()

  def write_part(nrow):
    ns = nrow // STRIP
    bufs = (zbuf, rows[0])
    row0 = cid * N + sid * RPT
    for t in range(ns):
      b = t % 2
      if t >= 2:
        pltpu.make_async_copy(bufs[b], acc_out.at[pl.ds(row0, STRIP)],
                              sems[b]).wait()
      pltpu.sync_copy(acc_sh.at[pl.ds(sid * RPT + t * STRIP, STRIP)],
                      bufs[b])
      pltpu.async_copy(bufs[b], acc_out.at[pl.ds(row0 + t * STRIP,
                                                 STRIP)], sems[b])
    for t in range(min(ns, 2)):
      pltpu.make_async_copy(bufs[t], acc_out.at[pl.ds(row0, STRIP)],
                            sems[t]).wait()

  _per_tile(sid, write_part)


_edge_agg = pl.kernel(
    _agg_body,
    out_type=[jax.ShapeDtypeStruct((NC * N, H), jnp.bfloat16)],
    mesh=_MESH,
    scratch_types=[
        pltpu.VMEM((STRIP, H), jnp.bfloat16),
        pltpu.VMEM((NCHUNK, CHUNK), jnp.int32),
        pltpu.VMEM((NCHUNK, CHUNK), jnp.int32),
        pltpu.VMEM_SHARED((N, H), jnp.bfloat16),
    ] + [pltpu.VMEM((CHUNK, H), jnp.bfloat16) for _ in range(NBUF)]
      + [pltpu.SemaphoreType.DMA for _ in range(NBUF)],
    compiler_params=_SC_PARAMS,
)


def _count_body(dst_hbm, z_cnt, ones_hbm, cnt_out, cbuf, dbuf, ones_v,
                cnt_sh, sem):
  cid = lax.axis_index("c")
  sid = lax.axis_index("s")

  def zero_part(nrow):
    pltpu.sync_copy(z_cnt, cbuf)
    _strip_loop(nrow, lambda r: pltpu.sync_copy(
        cbuf, cnt_sh.at[pl.ds(sid * RPT + r, STRIP)]))

  _per_tile(sid, zero_part)
  pltpu.sync_copy(dst_hbm.at[sid], dbuf)
  pltpu.sync_copy(ones_hbm, ones_v)
  plsc.subcore_barrier()

  j0 = cid * CHALF

  def fire(j, carry):
    pltpu.async_copy(ones_v, cnt_sh.at[dbuf.at[j0 + j]], sem, add=True)
    return carry

  lax.fori_loop(0, CHALF, fire, 0)

  def drain(j, carry):
    pltpu.make_async_copy(ones_v, cnt_sh.at[dbuf.at[j0]], sem).wait()
    return carry

  lax.fori_loop(0, CHALF, drain, 0)
  plsc.subcore_barrier()

  def write_part(nrow):
    def strip(r):
      pltpu.sync_copy(cnt_sh.at[pl.ds(sid * RPT + r, STRIP)], cbuf)
      pltpu.sync_copy(cbuf, cnt_out.at[pl.ds(cid * N + sid * RPT + r,
                                             STRIP)])
    _strip_loop(nrow, strip)

  _per_tile(sid, write_part)


_edge_count = pl.kernel(
    _count_body,
    out_type=[jax.ShapeDtypeStruct((NC * N, CW), jnp.float32)],
    mesh=_MESH,
    scratch_types=[
        pltpu.VMEM((STRIP, CW), jnp.float32),
        pltpu.VMEM((CPT, CHUNK), jnp.int32),
        pltpu.VMEM((CHUNK, CW), jnp.float32),
        pltpu.VMEM_SHARED((N, CW), jnp.float32),
        pltpu.SemaphoreType.DMA,
    ],
    compiler_params=_SC_PARAMS,
)


def _lin1_body(x_ref, w_ref, b_ref, o_ref):
  out = lax.dot_general(
      x_ref[...], w_ref[...], (((1,), (1,)), ((), ())),
      preferred_element_type=jnp.float32) + b_ref[...]
  o_ref[...] = out.astype(jnp.bfloat16)


def _sage_combine(pal_ref, par_ref, pcl_ref, pcr_ref, h_ref, wl_ref,
                  bl_ref, wr_ref):
  agg = pal_ref[...].astype(jnp.float32) + par_ref[...].astype(jnp.float32)
  cnt = pcl_ref[:, 0:1] + pcr_ref[:, 0:1]
  mean = agg / jnp.maximum(cnt, 1.0)
  h = h_ref[...].astype(jnp.float32)
  out = lax.dot_general(mean, wl_ref[...], (((1,), (1,)), ((), ())),
                        preferred_element_type=jnp.float32)
  out = out + bl_ref[...]
  out = out + lax.dot_general(h, wr_ref[...], (((1,), (1,)), ((), ())),
                              preferred_element_type=jnp.float32)
  norm = jnp.sqrt(jnp.sum(out * out, axis=1, keepdims=True))
  return out / jnp.maximum(norm, 1e-12)


def _combine_body(pal_ref, par_ref, pcl_ref, pcr_ref, h_ref, wl_ref,
                  bl_ref, wr_ref, o_ref):
  out = _sage_combine(pal_ref, par_ref, pcl_ref, pcr_ref, h_ref, wl_ref,
                      bl_ref, wr_ref)
  o_ref[...] = jnp.maximum(out, 0.0).astype(jnp.bfloat16)


def _final_body(nsteps, pal_ref, par_ref, pcl_ref, pcr_ref, h_ref, wl_ref,
                bl_ref, wr_ref, batch_ref, wout_ref, bout_ref, o_ref,
                gsum, gcnt):
  i = pl.program_id(0)
  h3 = _sage_combine(pal_ref, par_ref, pcl_ref, pcr_ref, h_ref, wl_ref,
                     bl_ref, wr_ref)

  bn = h3.shape[0]
  gid = lax.broadcasted_iota(jnp.int32, (G, bn), 0)
  onehot = jnp.where(gid == batch_ref[0], 1.0, 0.0)

  @pl.when(i == 0)
  def _():
    gsum[...] = jnp.zeros_like(gsum)
    gcnt[...] = jnp.zeros_like(gcnt)

  gsum[...] += lax.dot_general(onehot, h3, (((1,), (0,)), ((), ())),
                               preferred_element_type=jnp.float32)
  gcnt[...] += jnp.sum(onehot, axis=1, keepdims=True)

  @pl.when(i == nsteps - 1)
  def _():
    pooled = gsum[...] / jnp.maximum(gcnt[...], 1.0)
    logits = jnp.sum(pooled * wout_ref[...], axis=1, keepdims=True)
    o_ref[...] = jax.nn.sigmoid(logits + bout_ref[0, 0])


def kernel(x, edge_index, batch, W1, b1, c1_Wl, c1_bl, c1_Wr, c2_Wl,
           c2_bl, c2_Wr, Wout, bout):
  D = x.shape[1]
  H2 = c2_Wl.shape[0]
  src = edge_index[0]
  dst = edge_index[1]

  BN = N
  nsteps = N // BN
  row_spec = pl.BlockSpec((BN, H), lambda i: (i, 0))
  left_spec = pl.BlockSpec((BN, H), lambda i: (i, 0))
  right_spec = pl.BlockSpec((BN, H), lambda i: (i + nsteps, 0))
  cntl_spec = pl.BlockSpec((BN, CW), lambda i: (i, 0))
  cntr_spec = pl.BlockSpec((BN, CW), lambda i: (i + nsteps, 0))
  w_spec = lambda r, c: pl.BlockSpec((r, c), lambda i: (0, 0))

  src3 = src.reshape(NW, NCHUNK, CHUNK)
  dst3 = dst.reshape(NW, NCHUNK, CHUNK)
  dstc = dst.reshape(NS, CPT, CHUNK)
  z_rows = jnp.zeros((STRIP, H), jnp.bfloat16)
  z_cnt = jnp.zeros((STRIP, CW), jnp.float32)
  ones = jnp.ones((CHUNK, CW), jnp.float32)

  (cnt,) = _edge_count(dstc, z_cnt, ones)

  (h,) = pl.pallas_call(
      _lin1_body,
      grid=(nsteps,),
      in_specs=[
          pl.BlockSpec((BN, D), lambda i: (i, 0)),
          pl.BlockSpec((H, D), lambda i: (0, 0)),
          pl.BlockSpec((1, H), lambda i: (0, 0)),
      ],
      out_specs=[row_spec],
      out_shape=[jax.ShapeDtypeStruct((N, H), jnp.bfloat16)],
  )(x, W1, b1.reshape(1, H))

  (agg1,) = _edge_agg(h, src3, dst3, z_rows)

  (h2,) = pl.pallas_call(
      _combine_body,
      grid=(nsteps,),
      in_specs=[
          left_spec, right_spec, cntl_spec, cntr_spec, row_spec,
          w_spec(H, H), w_spec(1, H), w_spec(H, H),
      ],
      out_specs=[row_spec],
      out_shape=[jax.ShapeDtypeStruct((N, H), jnp.bfloat16)],
  )(agg1, agg1, cnt, cnt, h, c1_Wl, c1_bl.reshape(1, H), c1_Wr)

  (agg2,) = _edge_agg(h2, src3, dst3, z_rows)

  batchi = batch.reshape(nsteps, 1, BN)
  out = pl.pallas_call(
      functools.partial(_final_body, nsteps),
      grid=(nsteps,),
      in_specs=[
          left_spec, right_spec, cntl_spec, cntr_spec, row_spec,
          w_spec(H2, H), w_spec(1, H2), w_spec(H2, H),
          pl.BlockSpec((1, 1, BN), lambda i: (i, 0, 0)),
          w_spec(1, H2), w_spec(1, 1),
      ],
      out_specs=pl.BlockSpec((G, 1), lambda i: (0, 0)),
      out_shape=jax.ShapeDtypeStruct((G, 1), jnp.float32),
      scratch_shapes=[
          pltpu.VMEM((G, H2), jnp.float32),
          pltpu.VMEM((G, 1), jnp.float32),
      ],
  )(agg2, agg2, cnt, cnt, h2, c2_Wl, c2_bl.reshape(1, H2), c2_Wr,
    batchi, Wout, bout.reshape(1, 1))
  return out
